# baseline (speedup 1.0000x reference)
import functools
import jax
import jax.numpy as jnp
from jax import lax
from jax.experimental import pallas as pl
from jax.experimental.pallas import tpu as pltpu
from jax.experimental.pallas import tpu_sc as plsc

DIM = 64
SEQ = 200
BATCH = 4096
N = BATCH * SEQ
NC = 2
NS = 16
NW = NC * NS
RPW = N // NW
BLK = 128
NBLK = RPW // BLK
NSL = DIM // 16


def _sc_body(idx_hbm, table_hbm, pe_hbm, out_hbm, idx_v, pe_v, rows_v, sem):
    wid = lax.axis_index("s") * NC + lax.axis_index("c")
    base = wid * RPW
    pltpu.sync_copy(idx_hbm.at[pl.ds(base, RPW)], idx_v)
    pltpu.sync_copy(pe_hbm, pe_v)

    def block_body(b, s0):
        pltpu.async_copy(
            table_hbm.at[idx_v.at[pl.ds(b * BLK, BLK)]], rows_v, sem
        ).wait()

        def row_body(r, s):
            for c in range(NSL):
                sl = pl.ds(c * 16, 16)
                rows_v[r, sl] = rows_v[r, sl] * 8.0 + pe_v[s, sl]
            s = s + 1
            return jnp.where(s == SEQ, 0, s)

        s1 = lax.fori_loop(0, BLK, row_body, s0)
        pltpu.sync_copy(rows_v, out_hbm.at[pl.ds(base + b * BLK, BLK)])
        return s1

    lax.fori_loop(0, NBLK, block_body, 0)


_mesh = plsc.VectorSubcoreMesh(core_axis_name="c", subcore_axis_name="s")

_pe_call = functools.partial(
    pl.kernel,
    mesh=_mesh,
    out_type=jax.ShapeDtypeStruct((N, DIM), jnp.float32),
    scratch_types=[
        pltpu.VMEM((RPW,), jnp.int32),
        pltpu.VMEM((SEQ, DIM), jnp.float32),
        pltpu.VMEM((BLK, DIM), jnp.float32),
        pltpu.SemaphoreType.DMA,
    ],
    compiler_params=pltpu.CompilerParams(use_tc_tiling_on_sc=False),
)(_sc_body)


@jax.jit
def kernel(x, table, pe):
    idx = x.reshape(-1)
    pe_seq = pe[0, :SEQ, :]
    out = _pe_call(idx, table, pe_seq)
    return out.reshape(x.shape[0], x.shape[1], DIM)


# speedup vs baseline: 1.0974x; 1.0974x over previous
import functools
import jax
import jax.numpy as jnp
from jax import lax
from jax.experimental import pallas as pl
from jax.experimental.pallas import tpu as pltpu
from jax.experimental.pallas import tpu_sc as plsc

DIM = 64
SEQ = 200
BATCH = 4096
N = BATCH * SEQ
NC = 2
NS = 16
NW = NC * NS
RPW = N // NW
BLK = SEQ
NBLK = RPW // BLK
NBUF = 4
NSL = DIM // 16
G0 = 104
G1 = BLK - G0


def _sc_body(idx_hbm, table_hbm, pe_hbm, out_hbm,
             idx_v, pe_v, r0, r1, r2, r3, o0, o1,
             g0, g1, g2, g3, s0, s1):
    rows = (r0, r1, r2, r3)
    obuf = (o0, o1)
    gsem = (g0, g1, g2, g3)
    ssem = (s0, s1)
    wid = lax.axis_index("s") * NC + lax.axis_index("c")
    base = wid * RPW
    pltpu.sync_copy(idx_hbm.at[pl.ds(base, RPW)], idx_v)
    pltpu.sync_copy(pe_hbm, pe_v)

    def gather_descs(g, buf, sem):
        off = pl.multiple_of(g * BLK, 8)
        return (
            pltpu.make_async_copy(
                table_hbm.at[idx_v.at[pl.ds(off, G0)]], buf.at[pl.ds(0, G0)], sem),
            pltpu.make_async_copy(
                table_hbm.at[idx_v.at[pl.ds(off + G0, G1)]], buf.at[pl.ds(G0, G1)], sem),
        )

    for b in range(NBUF):
        for d in gather_descs(b, rows[b], gsem[b]):
            d.start()

    @pl.loop(0, NBLK // NBUF)
    def _outer(k):
        for b in range(NBUF):
            g = k * NBUF + b
            for d in gather_descs(g, rows[b], gsem[b]):
                d.wait()
            ob = obuf[b % 2]
            osem = ssem[b % 2]

            @pl.when(g >= 2)
            def _():
                pltpu.make_async_copy(ob, out_hbm.at[pl.ds(0, BLK)], osem).wait()

            rb = rows[b]

            @pl.loop(0, BLK, unroll=8)
            def _fma(r):
                for c in range(NSL):
                    sl = pl.ds(c * 16, 16)
                    ob[r, sl] = rb[r, sl] * 8.0 + pe_v[r, sl]

            pltpu.async_copy(ob, out_hbm.at[pl.ds(base + g * BLK, BLK)], osem)

            @pl.when(g + NBUF < NBLK)
            def _():
                for d in gather_descs(g + NBUF, rows[b], gsem[b]):
                    d.start()

    pltpu.make_async_copy(o0, out_hbm.at[pl.ds(0, BLK)], ssem[0]).wait()
    pltpu.make_async_copy(o1, out_hbm.at[pl.ds(0, BLK)], ssem[1]).wait()


_mesh = plsc.VectorSubcoreMesh(core_axis_name="c", subcore_axis_name="s")

_pe_call = functools.partial(
    pl.kernel,
    mesh=_mesh,
    out_type=jax.ShapeDtypeStruct((N, DIM), jnp.float32),
    scratch_types=[
        pltpu.VMEM((RPW,), jnp.int32),
        pltpu.VMEM((SEQ, DIM), jnp.float32),
        pltpu.VMEM((BLK, DIM), jnp.float32),
        pltpu.VMEM((BLK, DIM), jnp.float32),
        pltpu.VMEM((BLK, DIM), jnp.float32),
        pltpu.VMEM((BLK, DIM), jnp.float32),
        pltpu.VMEM((BLK, DIM), jnp.float32),
        pltpu.VMEM((BLK, DIM), jnp.float32),
        pltpu.SemaphoreType.DMA,
        pltpu.SemaphoreType.DMA,
        pltpu.SemaphoreType.DMA,
        pltpu.SemaphoreType.DMA,
        pltpu.SemaphoreType.DMA,
        pltpu.SemaphoreType.DMA,
    ],
    compiler_params=pltpu.CompilerParams(use_tc_tiling_on_sc=False),
)(_sc_body)


@jax.jit
def kernel(x, table, pe):
    idx = x.reshape(-1)
    pe_seq = pe[0, :SEQ, :]
    out = _pe_call(idx, table, pe_seq)
    return out.reshape(x.shape[0], x.shape[1], DIM)
